# baseline (speedup 1.0000x reference)
import functools

import jax
import jax.numpy as jnp
from jax import lax
from jax.experimental import pallas as pl
from jax.experimental.pallas import tpu as pltpu
from jax.experimental.pallas import tpu_sc as plsc

NUM_B = 256
N_ITERS = 6
NC = 2
NS = 16
NW = NC * NS
SEGS_PER = NUM_B // NW
CHUNK = 128
CHUNK_SHIFT = 7
LANES = 16
TILE = 512
NEG = -1e30


def _allsum(v):
    iota = lax.iota(jnp.int32, LANES)
    for step in (1, 2, 4, 8):
        v = v + v[jnp.bitwise_xor(iota, step)]
    return v


def _allmax(v):
    iota = lax.iota(jnp.int32, LANES)
    for step in (1, 2, 4, 8):
        v = jnp.maximum(v, v[jnp.bitwise_xor(iota, step)])
    return v


def _score_body(feat_ref, seg_ref, qh_ref, ql_ref, out_ref):
    tile, d = feat_ref.shape
    seg = seg_ref[0, 0, :]
    oh = (seg[:, None] == lax.broadcasted_iota(jnp.int32, (tile, NUM_B), 1))
    oh = oh.astype(jnp.bfloat16)
    qg = (jnp.dot(oh, qh_ref[...], preferred_element_type=jnp.float32)
          + jnp.dot(oh, ql_ref[...], preferred_element_type=jnp.float32))
    out_ref[0, 0, :] = jnp.sum(feat_ref[...] * qg, axis=1)


def _attn_body(feat_hbm, scores_hbm, offs_hbm, q_hbm, out_hbm,
               q_v, offs_v, buf0, wbuf, w_v, m_v, d_v, r_v, out_v,
               sem0, sem1, sem2, sem3):
    n_total, d = feat_hbm.shape
    groups = d // LANES
    cid = lax.axis_index("c")
    sid = lax.axis_index("s")
    wid = sid * NC + cid
    b0 = wid * SEGS_PER

    pltpu.sync_copy(offs_hbm.at[pl.ds(b0, 24)], offs_v)
    pltpu.sync_copy(q_hbm.at[pl.ds(b0, SEGS_PER)], q_v)

    zeros = jnp.zeros((LANES,), jnp.float32)

    def init_seg(k, _):
        m_v[k, pl.ds(0, LANES)] = jnp.full((LANES,), NEG, jnp.float32)
        d_v[k, pl.ds(0, LANES)] = zeros
        for j in range(groups):
            r_v[k, pl.ds(LANES * j, LANES)] = zeros
        return 0

    lax.fori_loop(0, SEGS_PER, init_seg, 0)

    head = offs_v[pl.ds(0, LANES)]
    row_lo = head[0]
    row_hi_v = offs_v[pl.ds(SEGS_PER, LANES)]
    row_hi = row_hi_v[0]
    base = (row_lo >> 3) << 3
    nch = (row_hi - base + (CHUNK - 1)) >> CHUNK_SHIFT

    def chunk_src(c):
        start = base + c * CHUNK
        start_c = jnp.minimum(start, n_total - CHUNK)
        start_c = pl.multiple_of(start_c, 8)
        return start, start_c

    fsems = (sem0, sem1)
    wsems = (sem2, sem3)

    def issue(c, p):
        _, sc = chunk_src(c)
        pltpu.async_copy(feat_hbm.at[pl.ds(sc, CHUNK)],
                         buf0.at[pl.ds(p * CHUNK, CHUNK)], fsems[p])
        pltpu.async_copy(scores_hbm.at[pl.ds(sc, CHUNK)],
                         wbuf.at[pl.ds(p * CHUNK, CHUNK)], wsems[p])

    def wait(c, p):
        _, sc = chunk_src(c)
        pltpu.make_async_copy(feat_hbm.at[pl.ds(sc, CHUNK)],
                              buf0.at[pl.ds(p * CHUNK, CHUNK)], fsems[p]).wait()
        pltpu.make_async_copy(scores_hbm.at[pl.ds(sc, CHUNK)],
                              wbuf.at[pl.ds(p * CHUNK, CHUNK)], wsems[p]).wait()

    for par in range(2):
        @pl.when(par < nch)
        def _(par=par):
            issue(par, par)

    lane_iota = lax.iota(jnp.int32, LANES)

    def process_chunk(c, bbase):
        start, start_c = chunk_src(c)

        def seg_body(k, _):
            ovec = offs_v[pl.ds(k, LANES)]
            rs = ovec[0]
            re = ovec[1]
            lo = jnp.maximum(rs, start)
            hi = jnp.minimum(re, start + CHUNK)

            @pl.when(lo < hi)
            def _():
                lo_rel = lo - start_c
                hi_rel = hi - start_c
                gstart = lo_rel >> 4
                gend = (hi_rel + (LANES - 1)) >> 4
                qreg = [q_v[k, pl.ds(LANES * j, LANES)] for j in range(groups)]
                m_old_vec = m_v[k, pl.ds(0, LANES)]
                lo_vec = jnp.full((LANES,), lo_rel, jnp.int32)
                hi_vec = jnp.full((LANES,), hi_rel, jnp.int32)

                def score_group(g, carry):
                    m_c, rowvec = carry
                    gb = g * LANES
                    sv = wbuf[pl.ds(bbase + gb, LANES)]
                    valid = (rowvec >= lo_vec) & (rowvec < hi_vec)
                    sv = jnp.where(valid, sv, NEG)
                    w_v[pl.ds(gb, LANES)] = sv
                    return jnp.maximum(m_c, sv), rowvec + LANES

                rv0 = lane_iota + (gstart << 4)
                m_lanes, _ = lax.fori_loop(
                    gstart, gend, score_group, (m_old_vec, rv0))
                m_new_vec = _allmax(m_lanes)

                scale = jnp.exp(m_old_vec - m_new_vec)
                d_acc = d_v[k, pl.ds(0, LANES)] * scale
                r_init = tuple(r_v[k, pl.ds(LANES * j, LANES)] * scale
                               for j in range(groups))

                def accum_group(g, carry):
                    gb = g * LANES
                    d_c = carry[0]
                    r_c = list(carry[1:])
                    wg = jnp.exp(w_v[pl.ds(gb, LANES)] - m_new_vec)
                    d_c = d_c + wg
                    for li in range(LANES):
                        bi = bbase + gb + li
                        a_vec = wg[jnp.full((LANES,), li, jnp.int32)]
                        for j in range(groups):
                            r_c[j] = r_c[j] + a_vec * buf0[bi, pl.ds(LANES * j, LANES)]
                    return (d_c,) + tuple(r_c)

                final = lax.fori_loop(gstart, gend, accum_group,
                                      (d_acc,) + r_init)
                d_v[k, pl.ds(0, LANES)] = final[0]
                for j in range(groups):
                    r_v[k, pl.ds(LANES * j, LANES)] = final[1 + j]
                m_v[k, pl.ds(0, LANES)] = m_new_vec

            return 0

        lax.fori_loop(0, SEGS_PER, seg_body, 0)

    def ring_body(c, _):
        par = c & 1
        for p in range(2):
            @pl.when(par == p)
            def _(p=p):
                wait(c, p)

        process_chunk(c, par << CHUNK_SHIFT)

        @pl.when(c + 2 < nch)
        def _():
            for p in range(2):
                @pl.when(par == p)
                def _(p=p):
                    issue(c + 2, p)

        return 0

    lax.fori_loop(0, nch, ring_body, 0)

    def fin(k, _):
        dv = _allsum(d_v[k, pl.ds(0, LANES)])
        inv = jnp.where(dv > 0.0, 1.0 / dv, 0.0)
        for j in range(groups):
            out_v[k, pl.ds(LANES * j, LANES)] = r_v[k, pl.ds(LANES * j, LANES)] * inv
        return 0

    lax.fori_loop(0, SEGS_PER, fin, 0)
    pltpu.sync_copy(out_v, out_hbm.at[pl.ds(b0, SEGS_PER)])


def _lstm_body(h_ref, c_ref, r_ref, a_ref, rw_ref, b_ref, h_out, c_out):
    d = h_ref.shape[1]
    h = h_ref[...]
    c = c_ref[...]
    r = r_ref[...]
    gates = (
        jnp.dot(h, a_ref[...], preferred_element_type=jnp.float32)
        + jnp.dot(r, rw_ref[...], preferred_element_type=jnp.float32)
        + b_ref[...]
    )
    i_g = jax.nn.sigmoid(gates[:, :d])
    f_g = jax.nn.sigmoid(gates[:, d:2 * d])
    g_g = jnp.tanh(gates[:, 2 * d:3 * d])
    o_g = jax.nn.sigmoid(gates[:, 3 * d:])
    c_new = f_g * c + i_g * g_g
    h_new = o_g * jnp.tanh(c_new)
    h_out[...] = h_new
    c_out[...] = c_new


def kernel(feat, segment_ids, W_ih, W_hh, b_ih, b_hh):
    n, d = feat.shape
    b = NUM_B
    ntiles = (n + TILE - 1) // TILE
    npad = ntiles * TILE

    offs = jnp.searchsorted(
        segment_ids, jnp.arange(b + 1, dtype=jnp.int32), side="left"
    ).astype(jnp.int32)
    offs = jnp.pad(offs, (0, 272 - (b + 1)), constant_values=n)

    feat_p = jnp.pad(feat, ((0, npad - n), (0, 0)))
    segs3 = jnp.pad(segment_ids, (0, npad - n), mode="edge")
    segs3 = segs3.reshape(ntiles, 1, TILE)

    w_ih_t = W_ih.T
    a_w = w_ih_t[:d] + W_hh.T
    r_w = w_ih_t[d:]
    bias = (b_ih + b_hh)[None, :]

    lstm = pl.pallas_call(
        _lstm_body,
        out_shape=(
            jax.ShapeDtypeStruct((b, d), jnp.float32),
            jax.ShapeDtypeStruct((b, d), jnp.float32),
        ),
    )

    score = pl.pallas_call(
        _score_body,
        grid=(ntiles,),
        in_specs=[
            pl.BlockSpec((TILE, d), lambda t: (t, 0)),
            pl.BlockSpec((1, 1, TILE), lambda t: (t, 0, 0)),
            pl.BlockSpec((b, d), lambda t: (0, 0)),
            pl.BlockSpec((b, d), lambda t: (0, 0)),
        ],
        out_specs=pl.BlockSpec((1, 1, TILE), lambda t: (t, 0, 0)),
        out_shape=jax.ShapeDtypeStruct((ntiles, 1, TILE), jnp.float32),
    )

    mesh = plsc.VectorSubcoreMesh(core_axis_name="c", subcore_axis_name="s")
    attn = functools.partial(
        pl.kernel,
        mesh=mesh,
        compiler_params=pltpu.CompilerParams(needs_layout_passes=False),
        out_type=jax.ShapeDtypeStruct((b, d), jnp.float32),
        scratch_types=[
            pltpu.VMEM((SEGS_PER, d), jnp.float32),
            pltpu.VMEM((24,), jnp.int32),
            pltpu.VMEM((2 * CHUNK, d), jnp.float32),
            pltpu.VMEM((2 * CHUNK,), jnp.float32),
            pltpu.VMEM((CHUNK,), jnp.float32),
            pltpu.VMEM((SEGS_PER, LANES), jnp.float32),
            pltpu.VMEM((SEGS_PER, LANES), jnp.float32),
            pltpu.VMEM((SEGS_PER, d), jnp.float32),
            pltpu.VMEM((SEGS_PER, d), jnp.float32),
            pltpu.SemaphoreType.DMA,
            pltpu.SemaphoreType.DMA,
            pltpu.SemaphoreType.DMA,
            pltpu.SemaphoreType.DMA,
        ],
    )(_attn_body)

    h = jnp.zeros((b, d), jnp.float32)
    c = jnp.zeros((b, d), jnp.float32)
    readout = jnp.zeros((b, d), jnp.float32)
    for _ in range(N_ITERS):
        h, c = lstm(h, c, readout, a_w, r_w, bias)
        q_hi = h.astype(jnp.bfloat16)
        q_lo = (h - q_hi.astype(jnp.float32)).astype(jnp.bfloat16)
        scores = score(feat_p, segs3, q_hi, q_lo).reshape(-1)
        readout = attn(feat, scores, offs, h)
    return jnp.concatenate([h, readout], axis=-1)


# speedup vs baseline: 1.1002x; 1.1002x over previous
import functools

import jax
import jax.numpy as jnp
from jax import lax
from jax.experimental import pallas as pl
from jax.experimental.pallas import tpu as pltpu
from jax.experimental.pallas import tpu_sc as plsc

NUM_B = 256
N_ITERS = 6
NC = 2
NS = 16
NW = NC * NS
SEGS_PER = NUM_B // NW
CHUNK = 128
CHUNK_SHIFT = 7
LANES = 16
TILE = 512
NEG = -1e30


def _allsum(v):
    iota = lax.iota(jnp.int32, LANES)
    for step in (1, 2, 4, 8):
        v = v + v[jnp.bitwise_xor(iota, step)]
    return v


def _allmax(v):
    iota = lax.iota(jnp.int32, LANES)
    for step in (1, 2, 4, 8):
        v = jnp.maximum(v, v[jnp.bitwise_xor(iota, step)])
    return v


def _score_body(feat_ref, seg_ref, q_ref, out_ref):
    tile, d = feat_ref.shape
    seg = seg_ref[0, 0, :]
    oh = (seg[:, None] == lax.broadcasted_iota(jnp.int32, (tile, NUM_B), 1))
    qg = jnp.dot(oh.astype(jnp.float32), q_ref[...],
                 preferred_element_type=jnp.float32)
    out_ref[0, 0, :] = jnp.sum(feat_ref[...] * qg, axis=1)


def _attn_body(feat_hbm, scores_hbm, offs_hbm, q_hbm, out_hbm,
               q_v, offs_v, buf0, wbuf, w_v, m_v, d_v, r_v, out_v,
               sem0, sem1, sem2, sem3):
    n_total, d = feat_hbm.shape
    groups = d // LANES
    cid = lax.axis_index("c")
    sid = lax.axis_index("s")
    wid = sid * NC + cid
    b0 = wid * SEGS_PER

    pltpu.sync_copy(offs_hbm.at[pl.ds(b0, 24)], offs_v)
    pltpu.sync_copy(q_hbm.at[pl.ds(b0, SEGS_PER)], q_v)

    zeros = jnp.zeros((LANES,), jnp.float32)

    def init_seg(k, _):
        m_v[k, pl.ds(0, LANES)] = jnp.full((LANES,), NEG, jnp.float32)
        d_v[k, pl.ds(0, LANES)] = zeros
        for j in range(groups):
            r_v[k, pl.ds(LANES * j, LANES)] = zeros
        return 0

    lax.fori_loop(0, SEGS_PER, init_seg, 0)

    head = offs_v[pl.ds(0, LANES)]
    row_lo = head[0]
    row_hi_v = offs_v[pl.ds(SEGS_PER, LANES)]
    row_hi = row_hi_v[0]
    base = (row_lo >> 3) << 3
    nch = (row_hi - base + (CHUNK - 1)) >> CHUNK_SHIFT

    def chunk_src(c):
        start = base + c * CHUNK
        start_c = jnp.minimum(start, n_total - CHUNK)
        start_c = pl.multiple_of(start_c, 8)
        return start, start_c

    fsems = (sem0, sem1)
    wsems = (sem2, sem3)

    def issue(c, p):
        _, sc = chunk_src(c)
        pltpu.async_copy(feat_hbm.at[pl.ds(sc, CHUNK)],
                         buf0.at[pl.ds(p * CHUNK, CHUNK)], fsems[p])
        pltpu.async_copy(scores_hbm.at[pl.ds(sc, CHUNK)],
                         wbuf.at[pl.ds(p * CHUNK, CHUNK)], wsems[p])

    def wait(c, p):
        _, sc = chunk_src(c)
        pltpu.make_async_copy(feat_hbm.at[pl.ds(sc, CHUNK)],
                              buf0.at[pl.ds(p * CHUNK, CHUNK)], fsems[p]).wait()
        pltpu.make_async_copy(scores_hbm.at[pl.ds(sc, CHUNK)],
                              wbuf.at[pl.ds(p * CHUNK, CHUNK)], wsems[p]).wait()

    for par in range(2):
        @pl.when(par < nch)
        def _(par=par):
            issue(par, par)

    lane_iota = lax.iota(jnp.int32, LANES)

    def process_chunk(c, bbase):
        start, start_c = chunk_src(c)

        def seg_body(k, _):
            ovec = offs_v[pl.ds(k, LANES)]
            rs = ovec[0]
            re = ovec[1]
            lo = jnp.maximum(rs, start)
            hi = jnp.minimum(re, start + CHUNK)

            @pl.when(lo < hi)
            def _():
                lo_rel = lo - start_c
                hi_rel = hi - start_c
                gstart = lo_rel >> 4
                gend = (hi_rel + (LANES - 1)) >> 4
                qreg = [q_v[k, pl.ds(LANES * j, LANES)] for j in range(groups)]
                m_old_vec = m_v[k, pl.ds(0, LANES)]
                lo_vec = jnp.full((LANES,), lo_rel, jnp.int32)
                hi_vec = jnp.full((LANES,), hi_rel, jnp.int32)

                def score_group(g, carry):
                    m_c, rowvec = carry
                    gb = g * LANES
                    sv = wbuf[pl.ds(bbase + gb, LANES)]
                    valid = (rowvec >= lo_vec) & (rowvec < hi_vec)
                    sv = jnp.where(valid, sv, NEG)
                    w_v[pl.ds(gb, LANES)] = sv
                    return jnp.maximum(m_c, sv), rowvec + LANES

                rv0 = lane_iota + (gstart << 4)
                m_lanes, _ = lax.fori_loop(
                    gstart, gend, score_group, (m_old_vec, rv0))
                m_new_vec = _allmax(m_lanes)

                scale = jnp.exp(m_old_vec - m_new_vec)
                d_acc = d_v[k, pl.ds(0, LANES)] * scale
                r_init = tuple(r_v[k, pl.ds(LANES * j, LANES)] * scale
                               for j in range(groups))

                def accum_group(g, carry):
                    gb = g * LANES
                    d_c = carry[0]
                    r_c = list(carry[1:])
                    wg = jnp.exp(w_v[pl.ds(gb, LANES)] - m_new_vec)
                    d_c = d_c + wg
                    for li in range(LANES):
                        bi = bbase + gb + li
                        a_vec = wg[jnp.full((LANES,), li, jnp.int32)]
                        for j in range(groups):
                            r_c[j] = r_c[j] + a_vec * buf0[bi, pl.ds(LANES * j, LANES)]
                    return (d_c,) + tuple(r_c)

                final = lax.fori_loop(gstart, gend, accum_group,
                                      (d_acc,) + r_init)
                d_v[k, pl.ds(0, LANES)] = final[0]
                for j in range(groups):
                    r_v[k, pl.ds(LANES * j, LANES)] = final[1 + j]
                m_v[k, pl.ds(0, LANES)] = m_new_vec

            return 0

        lax.fori_loop(0, SEGS_PER, seg_body, 0)

    def ring_body(c, _):
        par = c & 1
        for p in range(2):
            @pl.when(par == p)
            def _(p=p):
                wait(c, p)

        process_chunk(c, par << CHUNK_SHIFT)

        @pl.when(c + 2 < nch)
        def _():
            for p in range(2):
                @pl.when(par == p)
                def _(p=p):
                    issue(c + 2, p)

        return 0

    lax.fori_loop(0, nch, ring_body, 0)

    def fin(k, _):
        dv = _allsum(d_v[k, pl.ds(0, LANES)])
        inv = jnp.where(dv > 0.0, 1.0 / dv, 0.0)
        for j in range(groups):
            out_v[k, pl.ds(LANES * j, LANES)] = r_v[k, pl.ds(LANES * j, LANES)] * inv
        return 0

    lax.fori_loop(0, SEGS_PER, fin, 0)
    pltpu.sync_copy(out_v, out_hbm.at[pl.ds(b0, SEGS_PER)])


def _lstm_body(h_ref, c_ref, r_ref, a_ref, rw_ref, b_ref, h_out, c_out):
    d = h_ref.shape[1]
    h = h_ref[...]
    c = c_ref[...]
    r = r_ref[...]
    gates = (
        jnp.dot(h, a_ref[...], preferred_element_type=jnp.float32)
        + jnp.dot(r, rw_ref[...], preferred_element_type=jnp.float32)
        + b_ref[...]
    )
    i_g = jax.nn.sigmoid(gates[:, :d])
    f_g = jax.nn.sigmoid(gates[:, d:2 * d])
    g_g = jnp.tanh(gates[:, 2 * d:3 * d])
    o_g = jax.nn.sigmoid(gates[:, 3 * d:])
    c_new = f_g * c + i_g * g_g
    h_new = o_g * jnp.tanh(c_new)
    h_out[...] = h_new
    c_out[...] = c_new


def kernel(feat, segment_ids, W_ih, W_hh, b_ih, b_hh):
    n, d = feat.shape
    b = NUM_B
    ntiles = (n + TILE - 1) // TILE
    npad = ntiles * TILE

    offs = jnp.searchsorted(
        segment_ids, jnp.arange(b + 1, dtype=jnp.int32), side="left"
    ).astype(jnp.int32)
    offs = jnp.pad(offs, (0, 272 - (b + 1)), constant_values=n)

    feat_p = jnp.pad(feat, ((0, npad - n), (0, 0)))
    segs3 = jnp.pad(segment_ids, (0, npad - n), mode="edge")
    segs3 = segs3.reshape(ntiles, 1, TILE)

    w_ih_t = W_ih.T
    a_w = w_ih_t[:d] + W_hh.T
    r_w = w_ih_t[d:]
    bias = (b_ih + b_hh)[None, :]

    lstm = pl.pallas_call(
        _lstm_body,
        out_shape=(
            jax.ShapeDtypeStruct((b, d), jnp.float32),
            jax.ShapeDtypeStruct((b, d), jnp.float32),
        ),
    )

    score = pl.pallas_call(
        _score_body,
        grid=(ntiles,),
        in_specs=[
            pl.BlockSpec((TILE, d), lambda t: (t, 0)),
            pl.BlockSpec((1, 1, TILE), lambda t: (t, 0, 0)),
            pl.BlockSpec((b, d), lambda t: (0, 0)),
        ],
        out_specs=pl.BlockSpec((1, 1, TILE), lambda t: (t, 0, 0)),
        out_shape=jax.ShapeDtypeStruct((ntiles, 1, TILE), jnp.float32),
    )

    mesh = plsc.VectorSubcoreMesh(core_axis_name="c", subcore_axis_name="s")
    attn = functools.partial(
        pl.kernel,
        mesh=mesh,
        compiler_params=pltpu.CompilerParams(needs_layout_passes=False),
        out_type=jax.ShapeDtypeStruct((b, d), jnp.float32),
        scratch_types=[
            pltpu.VMEM((SEGS_PER, d), jnp.float32),
            pltpu.VMEM((24,), jnp.int32),
            pltpu.VMEM((2 * CHUNK, d), jnp.float32),
            pltpu.VMEM((2 * CHUNK,), jnp.float32),
            pltpu.VMEM((CHUNK,), jnp.float32),
            pltpu.VMEM((SEGS_PER, LANES), jnp.float32),
            pltpu.VMEM((SEGS_PER, LANES), jnp.float32),
            pltpu.VMEM((SEGS_PER, d), jnp.float32),
            pltpu.VMEM((SEGS_PER, d), jnp.float32),
            pltpu.SemaphoreType.DMA,
            pltpu.SemaphoreType.DMA,
            pltpu.SemaphoreType.DMA,
            pltpu.SemaphoreType.DMA,
        ],
    )(_attn_body)

    h = jnp.zeros((b, d), jnp.float32)
    c = jnp.zeros((b, d), jnp.float32)
    readout = jnp.zeros((b, d), jnp.float32)
    for _ in range(N_ITERS):
        h, c = lstm(h, c, readout, a_w, r_w, bias)
        scores = score(feat_p, segs3, h).reshape(-1)
        readout = attn(feat, scores, offs, h)
    return jnp.concatenate([h, readout], axis=-1)
